# baseline (speedup 1.0000x reference)
import functools

import jax
import jax.numpy as jnp
from jax import lax
from jax.experimental import pallas as pl
from jax.experimental.pallas import tpu as pltpu
from jax.experimental.pallas import tpu_sc as plsc

BATCH, SEQ, HIDDEN = 2, 2048, 2048
NUM_TABLE, CODE_LEN = 16, 12
TABLE_SIZE = 2 ** CODE_LEN
OUT = 1024
TOTAL_DIM = NUM_TABLE * CODE_LEN
N = BATCH * SEQ


TOK_BLK = 256


def _hash_body(x_ref, w_ref, b_ref, p_ref, m_ref, idx_ref, score_ref):
    z = jnp.dot(x_ref[...], w_ref[...], preferred_element_type=jnp.float32)
    z = z + b_ref[...]
    bits = (z > 0).astype(jnp.float32)
    za = jnp.abs(z)
    ls = -jnp.log(1.0 + jnp.exp(-za))
    codef = jnp.dot(bits, p_ref[...], preferred_element_type=jnp.float32)
    offs = lax.broadcasted_iota(jnp.int32, (1, NUM_TABLE), 1) * TABLE_SIZE
    idx_ref[...] = codef.astype(jnp.int32) + offs
    score_ref[...] = jnp.exp(
        jnp.dot(ls, m_ref[...], preferred_element_type=jnp.float32))


def _hash_call(x, w, b, tok_off, ntok):
    r = jnp.arange(TOTAL_DIM)[:, None]
    c = jnp.arange(NUM_TABLE)[None, :]
    blk = (r // CODE_LEN == c)
    p_mat = jnp.where(blk, (2.0 ** (r % CODE_LEN)), 0.0).astype(jnp.float32)
    m_mat = blk.astype(jnp.float32)
    blk_off = tok_off // TOK_BLK
    return pl.pallas_call(
        _hash_body,
        grid=(ntok // TOK_BLK,),
        in_specs=[
            pl.BlockSpec((TOK_BLK, HIDDEN), lambda i, o=blk_off: (o + i, 0)),
            pl.BlockSpec((HIDDEN, TOTAL_DIM), lambda i: (0, 0)),
            pl.BlockSpec((1, TOTAL_DIM), lambda i: (0, 0)),
            pl.BlockSpec((TOTAL_DIM, NUM_TABLE), lambda i: (0, 0)),
            pl.BlockSpec((TOTAL_DIM, NUM_TABLE), lambda i: (0, 0)),
        ],
        out_specs=[
            pl.BlockSpec((TOK_BLK, NUM_TABLE), lambda i: (i, 0)),
            pl.BlockSpec((TOK_BLK, NUM_TABLE), lambda i: (i, 0)),
        ],
        out_shape=[
            jax.ShapeDtypeStruct((ntok, NUM_TABLE), jnp.int32),
            jax.ShapeDtypeStruct((ntok, NUM_TABLE), jnp.float32),
        ],
    )(x, w, b, p_mat, m_mat)


NUM_WORKERS = 32
CT = 2
RPC = CT * NUM_TABLE


def _make_sc(ntok):
    tpw = ntok // NUM_WORKERS
    nchunk = tpw // CT

    def _sc_body(idx_hbm, score_hbm, tables_hbm, bias_hbm, out_hbm,
                 idx_v, score_v, bias_v, rows_v, outb_v,
                 gsem0, gsem1, osem0, osem1):
        gsems = (gsem0, gsem1)
        osems = (osem0, osem1)
        wid = lax.axis_index("s") * 2 + lax.axis_index("c")
        tok0 = wid * tpw
        ibase = tok0 * NUM_TABLE
        cp_i = pltpu.make_async_copy(
            idx_hbm.at[pl.ds(ibase, tpw * NUM_TABLE)], idx_v, osem0)
        cp_s = pltpu.make_async_copy(
            score_hbm.at[pl.ds(ibase, tpw * NUM_TABLE)], score_v, osem0)
        cp_b = pltpu.make_async_copy(bias_hbm, bias_v, osem0)
        cp_i.start()
        cp_s.start()
        cp_b.start()
        cp_i.wait()
        cp_s.wait()
        cp_b.wait()

        def start_gather(c, buf):
            pltpu.async_copy(
                tables_hbm.at[idx_v.at[pl.ds(c * RPC, RPC)]],
                rows_v.at[buf], gsems[buf])

        def wait_gather(buf):
            pltpu.make_async_copy(
                tables_hbm.at[idx_v.at[pl.ds(0, RPC)]],
                rows_v.at[buf], gsems[buf]).wait()

        def start_out(c, buf):
            pltpu.async_copy(
                outb_v.at[buf], out_hbm.at[pl.ds(tok0 + c * CT, CT)],
                osems[buf])

        def wait_out(buf):
            pltpu.make_async_copy(
                outb_v.at[buf], out_hbm.at[pl.ds(tok0, CT)],
                osems[buf]).wait()

        start_gather(0, 0)

        def chunk_step(c, buf):
            @pl.when(c + 1 < nchunk)
            def _():
                start_gather(c + 1, 1 - buf)
            wait_gather(buf)
            @pl.when(c >= 2)
            def _():
                wait_out(buf)
            for lt in range(CT):
                sbase = (c * CT + lt) * NUM_TABLE
                sv = score_v[pl.ds(sbase, NUM_TABLE)]
                sb = [jnp.full((16,), sv[t], jnp.float32)
                      for t in range(NUM_TABLE)]

                @plsc.parallel_loop(0, OUT, 16, unroll=4)
                def _dim_body(doff):
                    sl = pl.ds(doff, 16)
                    acc = bias_v[sl]
                    for t in range(NUM_TABLE):
                        acc = acc + rows_v[buf, lt * NUM_TABLE + t, sl] * sb[t]
                    outb_v[buf, lt, sl] = acc
            start_out(c, buf)

        def outer(g, _):
            chunk_step(g * 2, 0)
            chunk_step(g * 2 + 1, 1)
            return 0

        lax.fori_loop(0, nchunk // 2, outer, 0)
        wait_out(0)
        wait_out(1)

    mesh = plsc.VectorSubcoreMesh(core_axis_name="c", subcore_axis_name="s")
    return functools.partial(
        pl.kernel,
        out_type=jax.ShapeDtypeStruct((ntok, OUT), jnp.float32),
        mesh=mesh,
        scratch_types=[
            pltpu.VMEM((tpw * NUM_TABLE,), jnp.int32),
            pltpu.VMEM((tpw * NUM_TABLE,), jnp.float32),
            pltpu.VMEM((OUT,), jnp.float32),
            pltpu.VMEM((2, RPC, OUT), jnp.float32),
            pltpu.VMEM((2, CT, OUT), jnp.float32),
            pltpu.SemaphoreType.DMA,
            pltpu.SemaphoreType.DMA,
            pltpu.SemaphoreType.DMA,
            pltpu.SemaphoreType.DMA,
        ],
    )(_sc_body)


NSLICE = 2


def kernel(hidden_states, W_proj, b_proj, tables, bias):
    x = hidden_states.reshape(N, HIDDEN)
    b2 = b_proj.reshape(1, TOTAL_DIM)
    sl_tok = N // NSLICE
    sc = _make_sc(sl_tok)
    outs = []
    for i in range(NSLICE):
        idx, score = _hash_call(x, W_proj, b2, i * sl_tok, sl_tok)
        outs.append(sc(idx.reshape(-1), score.reshape(-1), tables, bias))
    out = jnp.concatenate(outs, axis=0)
    return out.reshape(BATCH, SEQ, OUT)


# speedup vs baseline: 1.2141x; 1.2141x over previous
import functools

import jax
import jax.numpy as jnp
from jax import lax
from jax.experimental import pallas as pl
from jax.experimental.pallas import tpu as pltpu
from jax.experimental.pallas import tpu_sc as plsc

BATCH, SEQ, HIDDEN = 2, 2048, 2048
NUM_TABLE, CODE_LEN = 16, 12
TABLE_SIZE = 2 ** CODE_LEN
OUT = 1024
TOTAL_DIM = NUM_TABLE * CODE_LEN
N = BATCH * SEQ


TOK_BLK = 256


def _hash_body(x_ref, w_ref, b_ref, p_ref, m_ref, idx_ref, score_ref):
    z = jnp.dot(x_ref[...], w_ref[...], preferred_element_type=jnp.float32)
    z = z + b_ref[...]
    bits = (z > 0).astype(jnp.float32)
    za = jnp.abs(z)
    ls = -jnp.log(1.0 + jnp.exp(-za))
    codef = jnp.dot(bits, p_ref[...], preferred_element_type=jnp.float32)
    offs = lax.broadcasted_iota(jnp.int32, (1, NUM_TABLE), 1) * TABLE_SIZE
    idx_ref[...] = codef.astype(jnp.int32) + offs
    score_ref[...] = jnp.exp(
        jnp.dot(ls, m_ref[...], preferred_element_type=jnp.float32))


def _hash_call(x, w, b, tok_off, ntok):
    r = jnp.arange(TOTAL_DIM)[:, None]
    c = jnp.arange(NUM_TABLE)[None, :]
    blk = (r // CODE_LEN == c)
    p_mat = jnp.where(blk, (2.0 ** (r % CODE_LEN)), 0.0).astype(jnp.float32)
    m_mat = blk.astype(jnp.float32)
    blk_off = tok_off // TOK_BLK
    return pl.pallas_call(
        _hash_body,
        grid=(ntok // TOK_BLK,),
        in_specs=[
            pl.BlockSpec((TOK_BLK, HIDDEN), lambda i, o=blk_off: (o + i, 0)),
            pl.BlockSpec((HIDDEN, TOTAL_DIM), lambda i: (0, 0)),
            pl.BlockSpec((1, TOTAL_DIM), lambda i: (0, 0)),
            pl.BlockSpec((TOTAL_DIM, NUM_TABLE), lambda i: (0, 0)),
            pl.BlockSpec((TOTAL_DIM, NUM_TABLE), lambda i: (0, 0)),
        ],
        out_specs=[
            pl.BlockSpec((TOK_BLK, NUM_TABLE), lambda i: (i, 0)),
            pl.BlockSpec((TOK_BLK, NUM_TABLE), lambda i: (i, 0)),
        ],
        out_shape=[
            jax.ShapeDtypeStruct((ntok, NUM_TABLE), jnp.int32),
            jax.ShapeDtypeStruct((ntok, NUM_TABLE), jnp.float32),
        ],
    )(x, w, b, p_mat, m_mat)


NUM_WORKERS = 32
CT = 2
RPC = CT * NUM_TABLE


def _make_sc(ntok):
    tpw = ntok // NUM_WORKERS
    nchunk = tpw // CT

    def _sc_body(idx_hbm, score_hbm, tables_hbm, bias_hbm, out_hbm,
                 idx_v, score_v, bias_v, rows_v, outb_v,
                 gsem0, gsem1, gsem2, osem0, osem1, osem2):
        gsems = (gsem0, gsem1, gsem2)
        osems = (osem0, osem1, osem2)
        wid = lax.axis_index("s") * 2 + lax.axis_index("c")
        tok0 = wid * tpw
        ibase = tok0 * NUM_TABLE
        cp_i = pltpu.make_async_copy(
            idx_hbm.at[pl.ds(ibase, tpw * NUM_TABLE)], idx_v, osem0)
        cp_s = pltpu.make_async_copy(
            score_hbm.at[pl.ds(ibase, tpw * NUM_TABLE)], score_v, osem0)
        cp_b = pltpu.make_async_copy(bias_hbm, bias_v, osem0)
        cp_i.start()
        cp_s.start()
        cp_b.start()
        cp_i.wait()
        cp_s.wait()
        cp_b.wait()

        def start_gather(c, buf):
            pltpu.async_copy(
                tables_hbm.at[idx_v.at[pl.ds(c * RPC, RPC)]],
                rows_v.at[buf], gsems[buf])

        def wait_gather(buf):
            pltpu.make_async_copy(
                tables_hbm.at[idx_v.at[pl.ds(0, RPC)]],
                rows_v.at[buf], gsems[buf]).wait()

        def start_out(c, buf):
            pltpu.async_copy(
                outb_v.at[buf], out_hbm.at[pl.ds(tok0 + c * CT, CT)],
                osems[buf])

        def wait_out(buf):
            pltpu.make_async_copy(
                outb_v.at[buf], out_hbm.at[pl.ds(tok0, CT)],
                osems[buf]).wait()

        start_gather(0, 0)
        start_gather(1, 1)

        def chunk_step(c, buf):
            @pl.when(c + 2 < nchunk)
            def _():
                start_gather(c + 2, (buf + 2) % 3)
            wait_gather(buf)
            @pl.when(c >= 3)
            def _():
                wait_out(buf)
            for lt in range(CT):
                sbase = (c * CT + lt) * NUM_TABLE
                sv = score_v[pl.ds(sbase, NUM_TABLE)]
                sb = [jnp.full((16,), sv[t], jnp.float32)
                      for t in range(NUM_TABLE)]

                @plsc.parallel_loop(0, OUT, 16, unroll=4)
                def _dim_body(doff):
                    sl = pl.ds(doff, 16)
                    acc = bias_v[sl]
                    for t in range(NUM_TABLE):
                        acc = acc + rows_v[buf, lt * NUM_TABLE + t, sl] * sb[t]
                    outb_v[buf, lt, sl] = acc
            start_out(c, buf)

        def outer(g, _):
            chunk_step(g * 3, 0)
            chunk_step(g * 3 + 1, 1)
            chunk_step(g * 3 + 2, 2)
            return 0

        lax.fori_loop(0, nchunk // 3, outer, 0)
        chunk_step(jnp.int32(nchunk - 1), (nchunk - 1) % 3)
        wait_out(0)
        wait_out(1)
        wait_out(2)

    mesh = plsc.VectorSubcoreMesh(core_axis_name="c", subcore_axis_name="s")
    return functools.partial(
        pl.kernel,
        out_type=jax.ShapeDtypeStruct((ntok, OUT), jnp.float32),
        mesh=mesh,
        scratch_types=[
            pltpu.VMEM((tpw * NUM_TABLE,), jnp.int32),
            pltpu.VMEM((tpw * NUM_TABLE,), jnp.float32),
            pltpu.VMEM((OUT,), jnp.float32),
            pltpu.VMEM((3, RPC, OUT), jnp.float32),
            pltpu.VMEM((3, CT, OUT), jnp.float32),
            pltpu.SemaphoreType.DMA,
            pltpu.SemaphoreType.DMA,
            pltpu.SemaphoreType.DMA,
            pltpu.SemaphoreType.DMA,
            pltpu.SemaphoreType.DMA,
            pltpu.SemaphoreType.DMA,
        ],
    )(_sc_body)


def kernel(hidden_states, W_proj, b_proj, tables, bias):
    x = hidden_states.reshape(N, HIDDEN)
    b2 = b_proj.reshape(1, TOTAL_DIM)
    idx, score = _hash_call(x, W_proj, b2, 0, N)
    out = _make_sc(N)(idx.reshape(-1), score.reshape(-1), tables, bias)
    return out.reshape(BATCH, SEQ, OUT)
